# baseline (device time: 58365 ns/iter reference)
import jax
import jax.numpy as jnp
from jax import lax
from jax.experimental import pallas as pl
from jax.experimental.pallas import tpu as pltpu

B, SQ, H, D = 2, 512, 8, 64
SCALE = D ** -0.5
LOG2E = 1.4426950408889634


def kernel(Q, K, V):
    qp = (Q.reshape(B * SQ, H * D) * (SCALE * LOG2E)).astype(jnp.bfloat16)
    kp = K.reshape(B * SQ, H * D).astype(jnp.bfloat16)
    vp = V.reshape(B * SQ, H * D).astype(jnp.bfloat16)

    def body(q_hbm, k_hbm, v_hbm, out_ref,
             q_ref, kl, vl, kr, vr,
             dma_sems, send_sems, recv_sems):
        my_x = lax.axis_index("x")
        my_y = lax.axis_index("y")
        my_z = lax.axis_index("z")
        partner = (1 - my_x, my_y, my_z)

        cp_k = pltpu.make_async_copy(k_hbm, kl, dma_sems.at[1])
        cp_v = pltpu.make_async_copy(v_hbm, vl, dma_sems.at[2])
        cp_q = pltpu.make_async_copy(q_hbm, q_ref, dma_sems.at[0])
        cp_k.start()
        cp_v.start()
        cp_q.start()

        barrier = pltpu.get_barrier_semaphore()
        pl.semaphore_signal(barrier, inc=1, device_id=partner,
                            device_id_type=pl.DeviceIdType.MESH)
        pl.semaphore_wait(barrier, 1)

        cp_k.wait()
        rdma_k = pltpu.make_async_remote_copy(
            src_ref=kl, dst_ref=kr,
            send_sem=send_sems.at[0], recv_sem=recv_sems.at[0],
            device_id=partner, device_id_type=pl.DeviceIdType.MESH)
        rdma_k.start()
        cp_v.wait()
        rdma_v = pltpu.make_async_remote_copy(
            src_ref=vl, dst_ref=vr,
            send_sem=send_sems.at[1], recv_sem=recv_sems.at[1],
            device_id=partner, device_id_type=pl.DeviceIdType.MESH)
        rdma_v.start()
        cp_q.wait()

        ones8 = jnp.ones((SQ, 8), jnp.bfloat16)

        def partial(q_h, k2d, v2d, b, h):
            ks = k2d[b * SQ:(b + 1) * SQ, h * D:(h + 1) * D]
            vs = v2d[b * SQ:(b + 1) * SQ, h * D:(h + 1) * D]
            s = lax.dot_general(q_h, ks, (((1,), (1,)), ((), ())),
                                preferred_element_type=jnp.float32)
            p = jnp.exp2(s.astype(jnp.bfloat16))
            l8 = lax.dot_general(p, ones8, (((1,), (0,)), ((), ())),
                                 preferred_element_type=jnp.float32)
            o = lax.dot_general(p, vs, (((1,), (0,)), ((), ())),
                                preferred_element_type=jnp.float32)
            return o, l8[:, :1]

        q_all = q_ref[...]
        k_loc, v_loc = kl[...], vl[...]
        partials = []
        for b in range(B):
            q_b = q_all[b * SQ:(b + 1) * SQ, :]
            for h in range(H):
                partials.append(
                    partial(q_b[:, h * D:(h + 1) * D], k_loc, v_loc, b, h))

        rdma_k.wait()
        rdma_v.wait()

        k_rem, v_rem = kr[...], vr[...]
        for b in range(B):
            q_b = q_all[b * SQ:(b + 1) * SQ, :]
            for h in range(H):
                o1, l1 = partials[b * H + h]
                o2, l2 = partial(q_b[:, h * D:(h + 1) * D],
                                 k_rem, v_rem, b, h)
                out_ref[b * SQ:(b + 1) * SQ, h * D:(h + 1) * D] = (
                    (o1 + o2) / (l1 + l2))

    out2d = pl.pallas_call(
        body,
        out_shape=jax.ShapeDtypeStruct((B * SQ, H * D), jnp.float32),
        in_specs=[pl.BlockSpec(memory_space=pl.ANY)] * 3,
        out_specs=pl.BlockSpec(memory_space=pltpu.VMEM),
        scratch_shapes=[
            pltpu.VMEM((B * SQ, H * D), jnp.bfloat16),
            pltpu.VMEM((B * SQ, H * D), jnp.bfloat16),
            pltpu.VMEM((B * SQ, H * D), jnp.bfloat16),
            pltpu.VMEM((B * SQ, H * D), jnp.bfloat16),
            pltpu.VMEM((B * SQ, H * D), jnp.bfloat16),
            pltpu.SemaphoreType.DMA((3,)),
            pltpu.SemaphoreType.DMA((2,)),
            pltpu.SemaphoreType.DMA((2,)),
        ],
        compiler_params=pltpu.CompilerParams(
            collective_id=0, vmem_limit_bytes=100 * 1024 * 1024),
    )(qp, kp, vp)
    return out2d.reshape(B, SQ, H, D)


# device time: 51735 ns/iter; 1.1282x vs baseline; 1.1282x over previous
import jax
import jax.numpy as jnp
from jax import lax
from jax.experimental import pallas as pl
from jax.experimental.pallas import tpu as pltpu

B, SQ, H, D = 2, 512, 8, 64
SCALE = D ** -0.5
LOG2E = 1.4426950408889634
NCHUNK = 4
CROWS = B * SQ // NCHUNK


def kernel(Q, K, V):
    qp = (Q.reshape(B * SQ, H * D) * (SCALE * LOG2E)).astype(jnp.bfloat16)
    kp = K.reshape(B * SQ, H * D).astype(jnp.bfloat16)
    vp = V.reshape(B * SQ, H * D).astype(jnp.bfloat16)

    def body(q_hbm, k_hbm, v_hbm, out_hbm,
             q2d, kl, vl, kr, vr, out_st,
             dma_sems, send_sems, recv_sems):
        my_x = lax.axis_index("x")
        my_y = lax.axis_index("y")
        my_z = lax.axis_index("z")
        p_dir = (1 - my_x, my_y, my_z)
        p_diag = (1 - my_x, 1 - my_y, my_z)

        cp_k = pltpu.make_async_copy(k_hbm, kl, dma_sems.at[1])
        cp_v = pltpu.make_async_copy(v_hbm, vl, dma_sems.at[2])
        cp_q = pltpu.make_async_copy(q_hbm, q2d, dma_sems.at[0])
        cp_k.start()
        cp_v.start()
        cp_q.start()

        barrier = pltpu.get_barrier_semaphore()
        for nbr in (p_dir, p_diag):
            pl.semaphore_signal(barrier, inc=1, device_id=nbr,
                                device_id_type=pl.DeviceIdType.MESH)
        pl.semaphore_wait(barrier, 2)

        def chunk_rdma(src, dst, i, target, sem_base):
            return pltpu.make_async_remote_copy(
                src_ref=src.at[pl.ds(i * CROWS, CROWS)],
                dst_ref=dst.at[pl.ds(i * CROWS, CROWS)],
                send_sem=send_sems.at[sem_base + i],
                recv_sem=recv_sems.at[sem_base + i],
                device_id=target, device_id_type=pl.DeviceIdType.MESH)

        cp_k.wait()
        rd_k = [chunk_rdma(kl, kr, i, p_dir, 0) for i in range(NCHUNK)]
        for r in rd_k:
            r.start()
        cp_v.wait()
        rd_v = [chunk_rdma(vl, vr, i, p_diag, NCHUNK) for i in range(NCHUNK)]
        for r in rd_v:
            r.start()
        cp_q.wait()

        def partial(q_h, ks, vs, nrows):
            ones8 = jnp.ones((nrows, 8), jnp.bfloat16)
            s = lax.dot_general(q_h, ks, (((1,), (1,)), ((), ())),
                                preferred_element_type=jnp.float32)
            p = jnp.exp2(s.astype(jnp.bfloat16))
            l8 = lax.dot_general(p, ones8, (((1,), (0,)), ((), ())),
                                 preferred_element_type=jnp.float32)
            o = lax.dot_general(p, vs, (((1,), (0,)), ((), ())),
                                preferred_element_type=jnp.float32)
            return o, l8[:, :1]

        q_all = q2d[...]
        k_loc, v_loc = kl[...], vl[...]
        acc = []
        for b in range(B):
            q_b = q_all[b * SQ:(b + 1) * SQ, :]
            for h in range(H):
                o, l = partial(q_b[:, h * D:(h + 1) * D],
                               k_loc[b * SQ:(b + 1) * SQ, h * D:(h + 1) * D],
                               v_loc[b * SQ:(b + 1) * SQ, h * D:(h + 1) * D],
                               SQ)
                acc.append([o, l])

        cp_out = []
        for i in range(NCHUNK):
            rd_k[i].wait()
            rd_v[i].wait()
            b = (i * CROWS) // SQ
            q_b = q_all[b * SQ:(b + 1) * SQ, :]
            k_rem = kr[pl.ds(i * CROWS, CROWS)]
            v_rem = vr[pl.ds(i * CROWS, CROWS)]
            for h in range(H):
                o, l = partial(q_b[:, h * D:(h + 1) * D],
                               k_rem[:, h * D:(h + 1) * D],
                               v_rem[:, h * D:(h + 1) * D],
                               CROWS)
                a = acc[b * H + h]
                a[0] = a[0] + o
                a[1] = a[1] + l
            if (i * CROWS + CROWS) % SQ == 0:
                for h in range(H):
                    o, l = acc[b * H + h]
                    out_st[b * SQ:(b + 1) * SQ, h * D:(h + 1) * D] = o / l
                cp = pltpu.make_async_copy(
                    out_st.at[pl.ds(b * SQ, SQ)],
                    out_hbm.at[pl.ds(b * SQ, SQ)],
                    dma_sems.at[3 + b])
                cp.start()
                cp_out.append(cp)
        for cp in cp_out:
            cp.wait()

    out2d = pl.pallas_call(
        body,
        out_shape=jax.ShapeDtypeStruct((B * SQ, H * D), jnp.float32),
        in_specs=[pl.BlockSpec(memory_space=pl.ANY)] * 3,
        out_specs=pl.BlockSpec(memory_space=pl.ANY),
        scratch_shapes=[
            pltpu.VMEM((B * SQ, H * D), jnp.bfloat16),
            pltpu.VMEM((B * SQ, H * D), jnp.bfloat16),
            pltpu.VMEM((B * SQ, H * D), jnp.bfloat16),
            pltpu.VMEM((B * SQ, H * D), jnp.bfloat16),
            pltpu.VMEM((B * SQ, H * D), jnp.bfloat16),
            pltpu.VMEM((B * SQ, H * D), jnp.float32),
            pltpu.SemaphoreType.DMA((5,)),
            pltpu.SemaphoreType.DMA((2 * NCHUNK,)),
            pltpu.SemaphoreType.DMA((2 * NCHUNK,)),
        ],
        compiler_params=pltpu.CompilerParams(
            collective_id=0, vmem_limit_bytes=100 * 1024 * 1024),
    )(qp, kp, vp)
    return out2d.reshape(B, SQ, H, D)


# device time: 49706 ns/iter; 1.1742x vs baseline; 1.0408x over previous
import jax
import jax.numpy as jnp
from jax import lax
from jax.experimental import pallas as pl
from jax.experimental.pallas import tpu as pltpu

B, SQ, H, D = 2, 512, 8, 64
SCALE = D ** -0.5
LOG2E = 1.4426950408889634


def kernel(Q, K, V):
    qp = (Q.reshape(B * SQ, H * D) * (SCALE * LOG2E)).astype(jnp.bfloat16)
    kp = K.reshape(B * SQ, H * D).astype(jnp.bfloat16)
    vp = V.reshape(B * SQ, H * D).astype(jnp.bfloat16)

    def body(q_hbm, k_hbm, v_hbm, out_hbm,
             q2d, kl, vl, kr, vr, out_st,
             dma_sems, send_sems, recv_sems):
        my_x = lax.axis_index("x")
        my_y = lax.axis_index("y")
        my_z = lax.axis_index("z")
        p_x = (1 - my_x, my_y, my_z)
        p_y = (my_x, 1 - my_y, my_z)

        lo = my_y * SQ
        hi = (1 - my_y) * SQ

        def stage(src, dst, s0, s1):
            c0 = pltpu.make_async_copy(
                src.at[pl.ds(lo, SQ)], dst.at[pl.ds(0, SQ)], dma_sems.at[s0])
            c1 = pltpu.make_async_copy(
                src.at[pl.ds(hi, SQ)], dst.at[pl.ds(SQ, SQ)], dma_sems.at[s1])
            c0.start()
            c1.start()
            return c0, c1

        cp_k = stage(k_hbm, kl, 0, 1)
        cp_v = stage(v_hbm, vl, 2, 3)
        cp_q = stage(q_hbm, q2d, 4, 5)

        barrier = pltpu.get_barrier_semaphore()
        for nbr in (p_x, p_y):
            pl.semaphore_signal(barrier, inc=1, device_id=nbr,
                                device_id_type=pl.DeviceIdType.MESH)
        pl.semaphore_wait(barrier, 2)

        def rdma(src, dst, rows_src, rows_dst, target, i):
            return pltpu.make_async_remote_copy(
                src_ref=src.at[pl.ds(rows_src, SQ)],
                dst_ref=dst.at[pl.ds(rows_dst, SQ)],
                send_sem=send_sems.at[i], recv_sem=recv_sems.at[i],
                device_id=target, device_id_type=pl.DeviceIdType.MESH)

        cp_k[0].wait()
        cp_k[1].wait()
        ph1_k = rdma(kl, kr, 0, 0, p_x, 0)
        ph1_k.start()
        cp_v[0].wait()
        cp_v[1].wait()
        ph1_v = rdma(vl, vr, 0, 0, p_x, 1)
        ph1_v.start()
        cp_q[0].wait()
        cp_q[1].wait()

        def partial(q_h, ks, vs):
            ones8 = jnp.ones((SQ, 8), jnp.bfloat16)
            s = lax.dot_general(q_h, ks, (((1,), (1,)), ((), ())),
                                preferred_element_type=jnp.float32)
            p = jnp.exp2(s.astype(jnp.bfloat16))
            l8 = lax.dot_general(p, ones8, (((1,), (0,)), ((), ())),
                                 preferred_element_type=jnp.float32)
            o = lax.dot_general(p, vs, (((1,), (0,)), ((), ())),
                                preferred_element_type=jnp.float32)
            return o, l8[:, :1]

        q_all = q2d[...]
        k_loc, v_loc = kl[...], vl[...]
        acc = []
        for bp in range(B):
            q_b = q_all[bp * SQ:(bp + 1) * SQ, :]
            for h in range(H):
                sl = slice(h * D, (h + 1) * D)
                o, l = partial(q_b[:, sl],
                               k_loc[bp * SQ:(bp + 1) * SQ, sl],
                               v_loc[bp * SQ:(bp + 1) * SQ, sl])
                acc.append([o, l])

        ph1_k.wait()
        fwd_k = rdma(kr, kr, 0, SQ, p_y, 2)
        fwd_k.start()
        ph1_v.wait()
        fwd_v = rdma(vr, vr, 0, SQ, p_y, 3)
        fwd_v.start()

        def remote_block(bp, out_sem):
            q_b = q_all[bp * SQ:(bp + 1) * SQ, :]
            k_rem = kr[pl.ds(bp * SQ, SQ)]
            v_rem = vr[pl.ds(bp * SQ, SQ)]
            for h in range(H):
                sl = slice(h * D, (h + 1) * D)
                o2, l2 = partial(q_b[:, sl], k_rem[:, sl], v_rem[:, sl])
                o1, l1 = acc[bp * H + h]
                out_st[bp * SQ:(bp + 1) * SQ, sl] = (o1 + o2) / (l1 + l2)
            dst_rows = lo if bp == 0 else hi
            cp = pltpu.make_async_copy(
                out_st.at[pl.ds(bp * SQ, SQ)],
                out_hbm.at[pl.ds(dst_rows, SQ)],
                dma_sems.at[6 + bp])
            cp.start()
            return cp

        cp_o0 = remote_block(0, 6)
        fwd_k.wait()
        fwd_v.wait()
        cp_o1 = remote_block(1, 7)
        cp_o0.wait()
        cp_o1.wait()

    out2d = pl.pallas_call(
        body,
        out_shape=jax.ShapeDtypeStruct((B * SQ, H * D), jnp.float32),
        in_specs=[pl.BlockSpec(memory_space=pl.ANY)] * 3,
        out_specs=pl.BlockSpec(memory_space=pl.ANY),
        scratch_shapes=[
            pltpu.VMEM((B * SQ, H * D), jnp.bfloat16),
            pltpu.VMEM((B * SQ, H * D), jnp.bfloat16),
            pltpu.VMEM((B * SQ, H * D), jnp.bfloat16),
            pltpu.VMEM((B * SQ, H * D), jnp.bfloat16),
            pltpu.VMEM((B * SQ, H * D), jnp.bfloat16),
            pltpu.VMEM((B * SQ, H * D), jnp.float32),
            pltpu.SemaphoreType.DMA((8,)),
            pltpu.SemaphoreType.DMA((4,)),
            pltpu.SemaphoreType.DMA((4,)),
        ],
        compiler_params=pltpu.CompilerParams(
            collective_id=0, vmem_limit_bytes=100 * 1024 * 1024),
    )(qp, kp, vp)
    return out2d.reshape(B, SQ, H, D)


# device time: 44759 ns/iter; 1.3040x vs baseline; 1.1105x over previous
import jax
import jax.numpy as jnp
from jax import lax
from jax.experimental import pallas as pl
from jax.experimental.pallas import tpu as pltpu

B, SQ, H, D = 2, 512, 8, 64
SCALE = D ** -0.5
LOG2E = 1.4426950408889634


def kernel(Q, K, V):
    qp = (Q.reshape(B * SQ, H * D) * (SCALE * LOG2E)).astype(jnp.bfloat16)
    kp = K.reshape(B * SQ, H * D).astype(jnp.bfloat16)
    vp = V.reshape(B * SQ, H * D).astype(jnp.bfloat16)

    def body(q_hbm, k_hbm, v_hbm, out_hbm,
             q2d, kl, vl, kr, vr, out_st,
             dma_sems, send_sems, recv_sems):
        my_x = lax.axis_index("x")
        my_y = lax.axis_index("y")
        my_z = lax.axis_index("z")
        p_x = (1 - my_x, my_y, my_z)
        p_y = (my_x, 1 - my_y, my_z)

        lo = my_y * SQ
        hi = (1 - my_y) * SQ

        def stage(src, dst, s0, s1):
            c0 = pltpu.make_async_copy(
                src.at[pl.ds(lo, SQ)], dst.at[pl.ds(0, SQ)], dma_sems.at[s0])
            c1 = pltpu.make_async_copy(
                src.at[pl.ds(hi, SQ)], dst.at[pl.ds(SQ, SQ)], dma_sems.at[s1])
            c0.start()
            c1.start()
            return c0, c1

        cp_k = stage(k_hbm, kl, 0, 1)
        cp_v = stage(v_hbm, vl, 2, 3)
        cp_q = stage(q_hbm, q2d, 4, 5)

        barrier = pltpu.get_barrier_semaphore()
        for nbr in (p_x, p_y):
            pl.semaphore_signal(barrier, inc=1, device_id=nbr,
                                device_id_type=pl.DeviceIdType.MESH)
        pl.semaphore_wait(barrier, 2)

        CR = SQ // 2

        def rdma(src, dst, rows_src, rows_dst, target, i):
            return pltpu.make_async_remote_copy(
                src_ref=src.at[pl.ds(rows_src, CR)],
                dst_ref=dst.at[pl.ds(rows_dst, CR)],
                send_sem=send_sems.at[i], recv_sem=recv_sems.at[i],
                device_id=target, device_id_type=pl.DeviceIdType.MESH)

        cp_k[0].wait()
        cp_k[1].wait()
        cp_v[0].wait()
        cp_v[1].wait()
        ph1 = []
        for c in range(2):
            for buf_l, buf_r, base in ((kl, kr, 0), (vl, vr, 4)):
                r = rdma(buf_l, buf_r, c * CR, c * CR, p_x, base + c)
                r.start()
                ph1.append(r)
        cp_q[0].wait()
        cp_q[1].wait()

        def partial(q_h, ks, vs, nrows=SQ):
            ones8 = jnp.ones((nrows, 8), jnp.bfloat16)
            s = lax.dot_general(q_h, ks, (((1,), (1,)), ((), ())),
                                preferred_element_type=jnp.float32)
            p = jnp.exp2(s.astype(jnp.bfloat16))
            l8 = lax.dot_general(p, ones8, (((1,), (0,)), ((), ())),
                                 preferred_element_type=jnp.float32)
            o = lax.dot_general(p, vs, (((1,), (0,)), ((), ())),
                                preferred_element_type=jnp.float32)
            return o, l8[:, :1]

        q_all = q2d[...]
        k_loc, v_loc = kl[...], vl[...]
        acc = []
        for bp in range(B):
            q_b = q_all[bp * SQ:(bp + 1) * SQ, :]
            for h in range(H):
                sl = slice(h * D, (h + 1) * D)
                o, l = partial(q_b[:, sl],
                               k_loc[bp * SQ:(bp + 1) * SQ, sl],
                               v_loc[bp * SQ:(bp + 1) * SQ, sl])
                acc.append([o, l])

        def remote_chunk(bp, c):
            q_b = q_all[bp * SQ:(bp + 1) * SQ, :]
            rows = bp * SQ + c * CR
            k_rem = kr[pl.ds(rows, CR)]
            v_rem = vr[pl.ds(rows, CR)]
            for h in range(H):
                sl = slice(h * D, (h + 1) * D)
                o2, l2 = partial(q_b[:, sl], k_rem[:, sl], v_rem[:, sl],
                                 CR)
                a = acc[bp * H + h]
                a[0] = a[0] + o2
                a[1] = a[1] + l2

        def finalize(bp):
            for h in range(H):
                sl = slice(h * D, (h + 1) * D)
                o, l = acc[bp * H + h]
                out_st[bp * SQ:(bp + 1) * SQ, sl] = (o / l).astype(
                    jnp.bfloat16)
            dst_rows = lo if bp == 0 else hi
            cp = pltpu.make_async_copy(
                out_st.at[pl.ds(bp * SQ, SQ)],
                out_hbm.at[pl.ds(dst_rows, SQ)],
                dma_sems.at[6 + bp])
            cp.start()
            return cp

        fwds = []
        for c in range(2):
            for i, (buf, base) in enumerate(((kr, 0), (vr, 4))):
                ph1[c * 2 + i].wait()
                f = rdma(buf, buf, c * CR, SQ + c * CR, p_y, base + 2 + c)
                f.start()
                fwds.append(f)
            remote_chunk(0, c)
        cp_o0 = finalize(0)

        for c in range(2):
            fwds[c * 2].wait()
            fwds[c * 2 + 1].wait()
            remote_chunk(1, c)
        cp_o1 = finalize(1)
        cp_o0.wait()
        cp_o1.wait()

    out2d = pl.pallas_call(
        body,
        out_shape=jax.ShapeDtypeStruct((B * SQ, H * D), jnp.bfloat16),
        in_specs=[pl.BlockSpec(memory_space=pl.ANY)] * 3,
        out_specs=pl.BlockSpec(memory_space=pl.ANY),
        scratch_shapes=[
            pltpu.VMEM((B * SQ, H * D), jnp.bfloat16),
            pltpu.VMEM((B * SQ, H * D), jnp.bfloat16),
            pltpu.VMEM((B * SQ, H * D), jnp.bfloat16),
            pltpu.VMEM((B * SQ, H * D), jnp.bfloat16),
            pltpu.VMEM((B * SQ, H * D), jnp.bfloat16),
            pltpu.VMEM((B * SQ, H * D), jnp.bfloat16),
            pltpu.SemaphoreType.DMA((8,)),
            pltpu.SemaphoreType.DMA((8,)),
            pltpu.SemaphoreType.DMA((8,)),
        ],
        compiler_params=pltpu.CompilerParams(
            collective_id=0, vmem_limit_bytes=100 * 1024 * 1024),
    )(qp, kp, vp)
    return out2d.reshape(B, SQ, H, D).astype(jnp.float32)


# device time: 43365 ns/iter; 1.3459x vs baseline; 1.0321x over previous
import jax
import jax.numpy as jnp
from jax import lax
from jax.experimental import pallas as pl
from jax.experimental.pallas import tpu as pltpu

B, SQ, H, D = 2, 512, 8, 64
SCALE = D ** -0.5
LOG2E = 1.4426950408889634


def kernel(Q, K, V):
    qp = (Q.reshape(B * SQ, H * D) * (SCALE * LOG2E)).astype(jnp.bfloat16)
    kp = K.reshape(B * SQ, H * D).astype(jnp.bfloat16)
    vp = V.reshape(B * SQ, H * D).astype(jnp.bfloat16)

    def body(q_hbm, k_hbm, v_hbm, out_hbm,
             q2d, kl, vl, kr, vr, out_st,
             dma_sems, send_sems, recv_sems):
        my_x = lax.axis_index("x")
        my_y = lax.axis_index("y")
        my_z = lax.axis_index("z")
        p_x = (1 - my_x, my_y, my_z)
        p_y = (my_x, 1 - my_y, my_z)

        lo = my_y * SQ
        hi = (1 - my_y) * SQ

        def stage(src, dst, s0, s1):
            c0 = pltpu.make_async_copy(
                src.at[pl.ds(lo, SQ)], dst.at[pl.ds(0, SQ)], dma_sems.at[s0])
            c1 = pltpu.make_async_copy(
                src.at[pl.ds(hi, SQ)], dst.at[pl.ds(SQ, SQ)], dma_sems.at[s1])
            c0.start()
            c1.start()
            return c0, c1

        cp_k = stage(k_hbm, kl, 0, 1)
        cp_v = stage(v_hbm, vl, 2, 3)
        cp_q = stage(q_hbm, q2d, 4, 5)

        barrier = pltpu.get_barrier_semaphore()
        for nbr in (p_x, p_y):
            pl.semaphore_signal(barrier, inc=1, device_id=nbr,
                                device_id_type=pl.DeviceIdType.MESH)
        pl.semaphore_wait(barrier, 2)

        CR = SQ // 2

        def rdma(src, dst, rows_src, rows_dst, target, i):
            return pltpu.make_async_remote_copy(
                src_ref=src.at[pl.ds(rows_src, CR)],
                dst_ref=dst.at[pl.ds(rows_dst, CR)],
                send_sem=send_sems.at[i], recv_sem=recv_sems.at[i],
                device_id=target, device_id_type=pl.DeviceIdType.MESH)

        cp_k[0].wait()
        cp_k[1].wait()
        cp_v[0].wait()
        cp_v[1].wait()
        ph1 = []
        for c in range(2):
            for buf_l, buf_r, base in ((kl, kr, 0), (vl, vr, 4)):
                r = rdma(buf_l, buf_r, c * CR, c * CR, p_x, base + c)
                r.start()
                ph1.append(r)
        cp_q[0].wait()
        cp_q[1].wait()

        def partial(q_h, ks, vs, nrows=SQ):
            s = lax.dot_general(q_h, ks, (((1,), (1,)), ((), ())),
                                preferred_element_type=jnp.float32)
            p = jnp.exp2(s.astype(jnp.bfloat16))
            va = jnp.concatenate(
                [vs, jnp.ones((nrows, D), jnp.bfloat16)], axis=1)
            oa = lax.dot_general(p, va, (((1,), (0,)), ((), ())),
                                 preferred_element_type=jnp.float32)
            return oa[:, :D], oa[:, D:D + 1]

        q_all = q2d[...]
        k_loc, v_loc = kl[...], vl[...]
        acc = []
        for bp in range(B):
            q_b = q_all[bp * SQ:(bp + 1) * SQ, :]
            for h in range(H):
                sl = slice(h * D, (h + 1) * D)
                o, l = partial(q_b[:, sl],
                               k_loc[bp * SQ:(bp + 1) * SQ, sl],
                               v_loc[bp * SQ:(bp + 1) * SQ, sl])
                acc.append([o, l])

        def remote_chunk(bp, c):
            q_b = q_all[bp * SQ:(bp + 1) * SQ, :]
            rows = bp * SQ + c * CR
            k_rem = kr[pl.ds(rows, CR)]
            v_rem = vr[pl.ds(rows, CR)]
            for h in range(H):
                sl = slice(h * D, (h + 1) * D)
                o2, l2 = partial(q_b[:, sl], k_rem[:, sl], v_rem[:, sl],
                                 CR)
                a = acc[bp * H + h]
                a[0] = a[0] + o2
                a[1] = a[1] + l2

        def finalize(bp):
            for h in range(H):
                sl = slice(h * D, (h + 1) * D)
                o, l = acc[bp * H + h]
                out_st[bp * SQ:(bp + 1) * SQ, sl] = (o / l).astype(
                    jnp.bfloat16)
            dst_rows = lo if bp == 0 else hi
            cp = pltpu.make_async_copy(
                out_st.at[pl.ds(bp * SQ, SQ)],
                out_hbm.at[pl.ds(dst_rows, SQ)],
                dma_sems.at[6 + bp])
            cp.start()
            return cp

        fwds = []
        for c in range(2):
            for i, (buf, base) in enumerate(((kr, 0), (vr, 4))):
                ph1[c * 2 + i].wait()
                f = rdma(buf, buf, c * CR, SQ + c * CR, p_y, base + 2 + c)
                f.start()
                fwds.append(f)
            remote_chunk(0, c)
        cp_o0 = finalize(0)

        for c in range(2):
            fwds[c * 2].wait()
            fwds[c * 2 + 1].wait()
            remote_chunk(1, c)
        cp_o1 = finalize(1)
        cp_o0.wait()
        cp_o1.wait()

    out2d = pl.pallas_call(
        body,
        out_shape=jax.ShapeDtypeStruct((B * SQ, H * D), jnp.bfloat16),
        in_specs=[pl.BlockSpec(memory_space=pl.ANY)] * 3,
        out_specs=pl.BlockSpec(memory_space=pl.ANY),
        scratch_shapes=[
            pltpu.VMEM((B * SQ, H * D), jnp.bfloat16),
            pltpu.VMEM((B * SQ, H * D), jnp.bfloat16),
            pltpu.VMEM((B * SQ, H * D), jnp.bfloat16),
            pltpu.VMEM((B * SQ, H * D), jnp.bfloat16),
            pltpu.VMEM((B * SQ, H * D), jnp.bfloat16),
            pltpu.VMEM((B * SQ, H * D), jnp.bfloat16),
            pltpu.SemaphoreType.DMA((8,)),
            pltpu.SemaphoreType.DMA((8,)),
            pltpu.SemaphoreType.DMA((8,)),
        ],
        compiler_params=pltpu.CompilerParams(
            collective_id=0, vmem_limit_bytes=100 * 1024 * 1024),
    )(qp, kp, vp)
    return out2d.reshape(B, SQ, H, D).astype(jnp.float32)
